# baseline (speedup 1.0000x reference)
import functools

import jax
import jax.numpy as jnp
from jax import lax
from jax.experimental import pallas as pl
from jax.experimental.pallas import tpu as pltpu
from jax.experimental.pallas import tpu_sc as plsc

EMBED = 32
OUT_D = 3 * EMBED
N0, N1, N2 = 22, 6, 2
NT = N0 * N1 * N2
E_TOTAL = 1600000

NC, NS, L = 2, 16, 16
NW = NC * NS
PER_W = E_TOTAL // NW
CHUNK = 400
NGRP = CHUNK // L
NSEG = 5
SEG = CHUNK // NSEG
NCHUNK = PER_W // CHUNK


def _build_table(W0, W1, W2):
    def body(w0_ref, w1_ref, w2_ref, t_ref):
        i = lax.broadcasted_iota(jnp.int32, (NT, 1), 0)
        oh0 = (i // (N1 * N2) == lax.broadcasted_iota(jnp.int32, (NT, N0), 1))
        oh1 = ((i // N2) % N1 == lax.broadcasted_iota(jnp.int32, (NT, N1), 1))
        oh2 = (i % N2 == lax.broadcasted_iota(jnp.int32, (NT, N2), 1))
        t0 = jnp.dot(oh0.astype(jnp.float32), w0_ref[:],
                     preferred_element_type=jnp.float32,
                     precision=lax.Precision.HIGHEST)
        t1 = jnp.dot(oh1.astype(jnp.float32), w1_ref[:],
                     preferred_element_type=jnp.float32,
                     precision=lax.Precision.HIGHEST)
        t2 = jnp.dot(oh2.astype(jnp.float32), w2_ref[:],
                     preferred_element_type=jnp.float32,
                     precision=lax.Precision.HIGHEST)
        t_ref[:] = jnp.concatenate([t0, t1, t2], axis=1)

    return pl.pallas_call(
        body,
        out_shape=jax.ShapeDtypeStruct((NT, OUT_D), jnp.float32),
    )(W0, W1, W2)


def _lane_gather(v, idx):
    dnums = lax.GatherDimensionNumbers(
        offset_dims=(), collapsed_slice_dims=(0,), start_index_map=(0,))
    return lax.gather(v, idx[:, None], dnums, (1,),
                      mode=lax.GatherScatterMode.PROMISE_IN_BOUNDS)


_mesh = plsc.VectorSubcoreMesh(core_axis_name="c", subcore_axis_name="s")


@functools.partial(
    pl.kernel,
    out_type=jax.ShapeDtypeStruct((E_TOTAL, OUT_D), jnp.float32),
    mesh=_mesh,
    compiler_params=pltpu.CompilerParams(use_tc_tiling_on_sc=False),
    scratch_types=[
        pltpu.VMEM((2, 3 * CHUNK), jnp.int32),
        pltpu.VMEM((2, NSEG, SEG), jnp.int32),
        pltpu.VMEM((2, CHUNK, OUT_D), jnp.float32),
        pltpu.SemaphoreType.DMA,
        pltpu.SemaphoreType.DMA,
        pltpu.SemaphoreType.DMA,
        pltpu.SemaphoreType.DMA,
        pltpu.SemaphoreType.DMA,
        pltpu.SemaphoreType.DMA,
    ],
)
def _sc_gather(attr_hbm, t_hbm, out_hbm,
               attr_v, idx_v, rows_v,
               asem0, asem1, gsem0, gsem1, wsem0, wsem1):
    wid = lax.axis_index("s") * NC + lax.axis_index("c")
    base0 = wid * PER_W
    asem = (asem0, asem1)
    gsem = (gsem0, gsem1)
    wsem = (wsem0, wsem1)

    def attr_copy(k, p):
        base3 = pl.multiple_of((base0 + k * CHUNK) * 3, 16)
        return pltpu.make_async_copy(
            attr_hbm.at[pl.ds(base3, 3 * CHUNK)], attr_v.at[p], asem[p])

    def gather_copies(p):
        return [
            pltpu.make_async_copy(
                t_hbm.at[idx_v.at[p, s]],
                rows_v.at[p, pl.ds(s * SEG, SEG)],
                gsem[p],
            )
            for s in range(NSEG)
        ]

    def write_copy(k, p):
        base = pl.multiple_of(base0 + k * CHUNK, 16)
        return pltpu.make_async_copy(
            rows_v.at[p], out_hbm.at[pl.ds(base, CHUNK)], wsem[p])

    iota = lax.iota(jnp.int32, L)
    sels = []
    for f in range(3):
        sel = iota * 3 + f
        sels.append((sel % L, sel >= L, sel >= 2 * L))

    def compute_idx(p):
        for g in range(NGRP):
            w = 3 * L * g
            r0 = attr_v[p, pl.ds(w, L)]
            r1 = attr_v[p, pl.ds(w + L, L)]
            r2 = attr_v[p, pl.ds(w + 2 * L, L)]

            def pick(f):
                selm, m1, m2 = sels[f]
                return jnp.where(
                    m2, _lane_gather(r2, selm),
                    jnp.where(m1, _lane_gather(r1, selm),
                              _lane_gather(r0, selm)))

            v0 = jnp.minimum(pick(0), N0 - 1)
            v1 = jnp.minimum(pick(1), N1 - 1)
            v2 = jnp.minimum(pick(2), N2 - 1)
            s, col = divmod(g * L, SEG)
            idx_v[p, s, pl.ds(col, L)] = v0 * (N1 * N2) + v1 * N2 + v2

    def process(k, p, prefetch_next, first_pair):
        attr_copy(k, p).wait()
        compute_idx(p)
        if not first_pair:
            write_copy(k, p).wait()
        for cp in gather_copies(p):
            cp.start()
        if prefetch_next:
            attr_copy(k + 1, 1 - p).start()
        if not (first_pair and p == 0):
            for cp in gather_copies(1 - p):
                cp.wait()
            write_copy(k - 1, 1 - p).start()

    attr_copy(0, 0).start()

    process(0, 0, True, True)
    process(1, 1, True, True)

    def pair_body(k2, carry):
        k = 2 * k2
        process(k, 0, True, False)
        process(k + 1, 1, True, False)
        return carry

    lax.fori_loop(1, NCHUNK // 2, pair_body, jnp.int32(0))

    k_last = NCHUNK - 1
    process(k_last, 0, False, False)
    for cp in gather_copies(0):
        cp.wait()
    write_copy(k_last, 0).start()
    write_copy(k_last - 1, 1).wait()
    write_copy(k_last, 0).wait()


def kernel(edge_attr, W0, W1, W2):
    table = _build_table(W0, W1, W2)
    attr_flat = edge_attr.reshape(-1)
    return _sc_gather(attr_flat, table)


# speedup vs baseline: 5.1587x; 5.1587x over previous
import functools

import jax
import jax.numpy as jnp
from jax import lax
from jax.experimental import pallas as pl
from jax.experimental.pallas import tpu as pltpu
from jax.experimental.pallas import tpu_sc as plsc

EMBED = 32
OUT_D = 3 * EMBED
N0, N1, N2 = 22, 6, 2
NT = N0 * N1 * N2
E_TOTAL = 1600000

NC, NS, L = 2, 16, 16
NW = NC * NS
PER_W = E_TOTAL // NW
CHUNK = 400
NGRP = CHUNK // L
NSEG = 5
SEG = CHUNK // NSEG
NCHUNK = PER_W // CHUNK


def _build_table(W0, W1, W2):
    def body(w0_ref, w1_ref, w2_ref, t_ref):
        i = lax.broadcasted_iota(jnp.int32, (NT, 1), 0)
        oh0 = (i // (N1 * N2) == lax.broadcasted_iota(jnp.int32, (NT, N0), 1))
        oh1 = ((i // N2) % N1 == lax.broadcasted_iota(jnp.int32, (NT, N1), 1))
        oh2 = (i % N2 == lax.broadcasted_iota(jnp.int32, (NT, N2), 1))
        t0 = jnp.dot(oh0.astype(jnp.float32), w0_ref[:],
                     preferred_element_type=jnp.float32,
                     precision=lax.Precision.HIGHEST)
        t1 = jnp.dot(oh1.astype(jnp.float32), w1_ref[:],
                     preferred_element_type=jnp.float32,
                     precision=lax.Precision.HIGHEST)
        t2 = jnp.dot(oh2.astype(jnp.float32), w2_ref[:],
                     preferred_element_type=jnp.float32,
                     precision=lax.Precision.HIGHEST)
        t_ref[:] = jnp.concatenate([t0, t1, t2], axis=1)

    return pl.pallas_call(
        body,
        out_shape=jax.ShapeDtypeStruct((NT, OUT_D), jnp.float32),
    )(W0, W1, W2)


_mesh = plsc.VectorSubcoreMesh(core_axis_name="c", subcore_axis_name="s")


@functools.partial(
    pl.kernel,
    out_type=jax.ShapeDtypeStruct((E_TOTAL, OUT_D), jnp.float32),
    mesh=_mesh,
    compiler_params=pltpu.CompilerParams(use_tc_tiling_on_sc=False),
    scratch_types=[
        pltpu.VMEM((2, 3, CHUNK), jnp.int32),
        pltpu.VMEM((2, NSEG, SEG), jnp.int32),
        pltpu.VMEM((2, CHUNK, OUT_D), jnp.float32),
        pltpu.VMEM_SHARED((NT, OUT_D), jnp.float32),
        pltpu.SemaphoreType.DMA,
        pltpu.SemaphoreType.DMA,
        pltpu.SemaphoreType.DMA,
        pltpu.SemaphoreType.DMA,
        pltpu.SemaphoreType.DMA,
        pltpu.SemaphoreType.DMA,
    ],
)
def _sc_gather(attr_hbm, t_hbm, out_hbm,
               attr_v, idx_v, rows_v, t_sh, asem0, asem1, gsem0, gsem1, wsem0, wsem1):
    wid = lax.axis_index("s") * NC + lax.axis_index("c")
    base0 = wid * PER_W

    @pl.when(lax.axis_index("s") == 0)
    def _stage_table():
        pltpu.sync_copy(t_hbm, t_sh)

    plsc.subcore_barrier()

    asem = (asem0, asem1)
    gsem = (gsem0, gsem1)
    wsem = (wsem0, wsem1)

    def attr_copy(k, p):
        base = pl.multiple_of(base0 + k * CHUNK, 16)
        return pltpu.make_async_copy(
            attr_hbm.at[:, pl.ds(base, CHUNK)], attr_v.at[p], asem[p])

    def gather_copies(p):
        return [
            pltpu.make_async_copy(
                t_sh.at[idx_v.at[p, s]],
                rows_v.at[p, pl.ds(s * SEG, SEG)],
                gsem[p],
            )
            for s in range(NSEG)
        ]

    def write_copy(k, p):
        base = pl.multiple_of(base0 + k * CHUNK, 16)
        return pltpu.make_async_copy(
            rows_v.at[p], out_hbm.at[pl.ds(base, CHUNK)], wsem[p])

    def compute_idx(p):
        for g in range(NGRP):
            s, col = divmod(g * L, SEG)
            v0 = jnp.minimum(attr_v[p, 0, pl.ds(g * L, L)], N0 - 1)
            v1 = jnp.minimum(attr_v[p, 1, pl.ds(g * L, L)], N1 - 1)
            v2 = jnp.minimum(attr_v[p, 2, pl.ds(g * L, L)], N2 - 1)
            idx_v[p, s, pl.ds(col, L)] = v0 * (N1 * N2) + v1 * N2 + v2

    def process(k, p, prefetch_next, first_pair):
        attr_copy(k, p).wait()
        compute_idx(p)
        if not first_pair:
            write_copy(k, p).wait()
        for cp in gather_copies(p):
            cp.start()
        if prefetch_next:
            attr_copy(k + 1, 1 - p).start()
        if not (first_pair and p == 0):
            for cp in gather_copies(1 - p):
                cp.wait()
            write_copy(k - 1, 1 - p).start()

    attr_copy(0, 0).start()

    process(0, 0, True, True)
    process(1, 1, True, True)

    def pair_body(k2, carry):
        k = 2 * k2
        process(k, 0, True, False)
        process(k + 1, 1, True, False)
        return carry

    lax.fori_loop(1, NCHUNK // 2, pair_body, jnp.int32(0))

    k_last = NCHUNK - 1
    process(k_last, 0, False, False)
    for cp in gather_copies(0):
        cp.wait()
    write_copy(k_last, 0).start()
    write_copy(k_last - 1, 1).wait()
    write_copy(k_last, 0).wait()


def kernel(edge_attr, W0, W1, W2):
    table = _build_table(W0, W1, W2)
    attr_t = edge_attr.T
    return _sc_gather(attr_t, table)


# speedup vs baseline: 6.0519x; 1.1731x over previous
import functools

import jax
import jax.numpy as jnp
from jax import lax
from jax.experimental import pallas as pl
from jax.experimental.pallas import tpu as pltpu
from jax.experimental.pallas import tpu_sc as plsc

EMBED = 32
OUT_D = 3 * EMBED
N0, N1, N2 = 22, 6, 2
NT = N0 * N1 * N2
E_TOTAL = 1600000

NC, NS, L = 2, 16, 16
NW = NC * NS
PER_W = E_TOTAL // NW
CHUNK = 400
NGRP = CHUNK // L
NSEG = 5
SEG = CHUNK // NSEG
NCHUNK = PER_W // CHUNK


def _build_table(W0, W1, W2):
    def body(w0_ref, w1_ref, w2_ref, t_ref):
        i = lax.broadcasted_iota(jnp.int32, (NT, 1), 0)
        oh0 = (i // (N1 * N2) == lax.broadcasted_iota(jnp.int32, (NT, N0), 1))
        oh1 = ((i // N2) % N1 == lax.broadcasted_iota(jnp.int32, (NT, N1), 1))
        oh2 = (i % N2 == lax.broadcasted_iota(jnp.int32, (NT, N2), 1))
        t0 = jnp.dot(oh0.astype(jnp.float32), w0_ref[:],
                     preferred_element_type=jnp.float32,
                     precision=lax.Precision.HIGHEST)
        t1 = jnp.dot(oh1.astype(jnp.float32), w1_ref[:],
                     preferred_element_type=jnp.float32,
                     precision=lax.Precision.HIGHEST)
        t2 = jnp.dot(oh2.astype(jnp.float32), w2_ref[:],
                     preferred_element_type=jnp.float32,
                     precision=lax.Precision.HIGHEST)
        t_ref[:] = jnp.concatenate([t0, t1, t2], axis=1)

    return pl.pallas_call(
        body,
        out_shape=jax.ShapeDtypeStruct((NT, OUT_D), jnp.float32),
    )(W0, W1, W2)


_mesh = plsc.VectorSubcoreMesh(core_axis_name="c", subcore_axis_name="s")


@functools.partial(
    pl.kernel,
    out_type=jax.ShapeDtypeStruct((E_TOTAL, OUT_D), jnp.float32),
    mesh=_mesh,
    compiler_params=pltpu.CompilerParams(use_tc_tiling_on_sc=False),
    scratch_types=[
        pltpu.VMEM((2, 3, CHUNK), jnp.int32),
        pltpu.VMEM((2, NSEG, SEG), jnp.int32),
        pltpu.VMEM((2, CHUNK, OUT_D), jnp.float32),
        pltpu.VMEM_SHARED((NT, OUT_D), jnp.float32),
        pltpu.SemaphoreType.DMA,
        pltpu.SemaphoreType.DMA,
        pltpu.SemaphoreType.DMA,
        pltpu.SemaphoreType.DMA,
        pltpu.SemaphoreType.DMA,
        pltpu.SemaphoreType.DMA,
    ],
)
def _sc_gather(a0_hbm, a1_hbm, a2_hbm, t_hbm, out_hbm,
               attr_v, idx_v, rows_v, t_sh, asem0, asem1, gsem0, gsem1, wsem0, wsem1):
    wid = lax.axis_index("s") * NC + lax.axis_index("c")
    base0 = wid * PER_W

    @pl.when(lax.axis_index("s") == 0)
    def _stage_table():
        pltpu.sync_copy(t_hbm, t_sh)

    plsc.subcore_barrier()

    asem = (asem0, asem1)
    gsem = (gsem0, gsem1)
    wsem = (wsem0, wsem1)

    def attr_copies(k, p):
        base = pl.multiple_of(base0 + k * CHUNK, 16)
        return [pltpu.make_async_copy(
            a_hbm.at[pl.ds(base, CHUNK)], attr_v.at[p, f], asem[p])
            for f, a_hbm in enumerate((a0_hbm, a1_hbm, a2_hbm))]

    def gather_copies(p):
        return [
            pltpu.make_async_copy(
                t_sh.at[idx_v.at[p, s]],
                rows_v.at[p, pl.ds(s * SEG, SEG)],
                gsem[p],
            )
            for s in range(NSEG)
        ]

    def write_copy(k, p):
        base = pl.multiple_of(base0 + k * CHUNK, 16)
        return pltpu.make_async_copy(
            rows_v.at[p], out_hbm.at[pl.ds(base, CHUNK)], wsem[p])

    def compute_idx(p):
        for g in range(NGRP):
            s, col = divmod(g * L, SEG)
            v0 = jnp.minimum(attr_v[p, 0, pl.ds(g * L, L)], N0 - 1)
            v1 = jnp.minimum(attr_v[p, 1, pl.ds(g * L, L)], N1 - 1)
            v2 = jnp.minimum(attr_v[p, 2, pl.ds(g * L, L)], N2 - 1)
            idx_v[p, s, pl.ds(col, L)] = v0 * (N1 * N2) + v1 * N2 + v2

    def process(k, p, prefetch_next, first_pair):
        for cp in attr_copies(k, p):
            cp.wait()
        compute_idx(p)
        if not first_pair:
            write_copy(k, p).wait()
        for cp in gather_copies(p):
            cp.start()
        if prefetch_next:
            for cp in attr_copies(k + 1, 1 - p):
                cp.start()
        if not (first_pair and p == 0):
            for cp in gather_copies(1 - p):
                cp.wait()
            write_copy(k - 1, 1 - p).start()

    for _cp in attr_copies(0, 0):
        _cp.start()

    process(0, 0, True, True)
    process(1, 1, True, True)

    def pair_body(k2, carry):
        k = 2 * k2
        process(k, 0, True, False)
        process(k + 1, 1, True, False)
        return carry

    lax.fori_loop(1, NCHUNK // 2, pair_body, jnp.int32(0))

    k_last = NCHUNK - 1
    process(k_last, 0, False, False)
    for cp in gather_copies(0):
        cp.wait()
    write_copy(k_last, 0).start()
    write_copy(k_last - 1, 1).wait()
    write_copy(k_last, 0).wait()


def kernel(edge_attr, W0, W1, W2):
    table = _build_table(W0, W1, W2)
    a0 = edge_attr[:, 0]
    a1 = edge_attr[:, 1]
    a2 = edge_attr[:, 2]
    return _sc_gather(a0, a1, a2, table)


# speedup vs baseline: 6.8936x; 1.1391x over previous
import functools

import jax
import jax.numpy as jnp
from jax import lax
from jax.experimental import pallas as pl
from jax.experimental.pallas import tpu as pltpu
from jax.experimental.pallas import tpu_sc as plsc

EMBED = 32
OUT_D = 3 * EMBED
PAD_D = 128
N0, N1, N2 = 22, 6, 2
NT = N0 * N1 * N2
E_TOTAL = 1600000

NC, NS, L = 2, 16, 16
NW = NC * NS
PER_W = E_TOTAL // NW
CHUNK = 400
NGRP = CHUNK // L
NSEG = 5
SEG = CHUNK // NSEG
NCHUNK = PER_W // CHUNK
NK = OUT_D // L


def _build_table(W0, W1, W2):
    def body(w0_ref, w1_ref, w2_ref, t_ref):
        i = lax.broadcasted_iota(jnp.int32, (NT, 1), 0)
        oh0 = (i // (N1 * N2) == lax.broadcasted_iota(jnp.int32, (NT, N0), 1))
        oh1 = ((i // N2) % N1 == lax.broadcasted_iota(jnp.int32, (NT, N1), 1))
        oh2 = (i % N2 == lax.broadcasted_iota(jnp.int32, (NT, N2), 1))
        t0 = jnp.dot(oh0.astype(jnp.float32), w0_ref[:],
                     preferred_element_type=jnp.float32,
                     precision=lax.Precision.HIGHEST)
        t1 = jnp.dot(oh1.astype(jnp.float32), w1_ref[:],
                     preferred_element_type=jnp.float32,
                     precision=lax.Precision.HIGHEST)
        t2 = jnp.dot(oh2.astype(jnp.float32), w2_ref[:],
                     preferred_element_type=jnp.float32,
                     precision=lax.Precision.HIGHEST)
        pad = jnp.zeros((NT, PAD_D - OUT_D), jnp.float32)
        t_ref[:] = jnp.concatenate([t0, t1, t2, pad], axis=1)

    return pl.pallas_call(
        body,
        out_shape=jax.ShapeDtypeStruct((NT, PAD_D), jnp.float32),
    )(W0, W1, W2)


_mesh = plsc.VectorSubcoreMesh(core_axis_name="c", subcore_axis_name="s")


@functools.partial(
    pl.kernel,
    out_type=jax.ShapeDtypeStruct((E_TOTAL, OUT_D), jnp.float32),
    mesh=_mesh,
    scratch_types=[
        pltpu.VMEM((2, 1, CHUNK), jnp.int32),
        pltpu.VMEM((2, 1, CHUNK), jnp.int32),
        pltpu.VMEM((2, 1, CHUNK), jnp.int32),
        pltpu.VMEM((2, NSEG, 1, SEG), jnp.int32),
        pltpu.VMEM((2, SEG, PAD_D), jnp.float32),
        pltpu.VMEM((2, CHUNK, OUT_D), jnp.float32),
        pltpu.VMEM_SHARED((NT, PAD_D), jnp.float32),
        pltpu.SemaphoreType.DMA,
        pltpu.SemaphoreType.DMA,
        pltpu.SemaphoreType.DMA,
        pltpu.SemaphoreType.DMA,
        pltpu.SemaphoreType.DMA,
        pltpu.SemaphoreType.DMA,
    ],
)
def _sc_gather(a0_hbm, a1_hbm, a2_hbm, t_hbm, out_hbm,
               a0_v, a1_v, a2_v, idx_v, seg_v, rows_v, t_sh,
               asem0, asem1, gsem0, gsem1, wsem0, wsem1):
    wid = lax.axis_index("s") * NC + lax.axis_index("c")
    base0 = wid * PER_W

    @pl.when(lax.axis_index("s") == 0)
    def _stage_table():
        pltpu.sync_copy(t_hbm, t_sh)

    plsc.subcore_barrier()

    asem = (asem0, asem1)
    gsem = (gsem0, gsem1)
    wsem = (wsem0, wsem1)

    def attr_copies(k, p):
        base = pl.multiple_of(base0 + k * CHUNK, 16)
        srcs = (a0_hbm, a1_hbm, a2_hbm)
        dsts = (a0_v, a1_v, a2_v)
        return [pltpu.make_async_copy(
            srcs[f].at[pl.ds(base, CHUNK)], dsts[f].at[p, 0], asem[p])
            for f in range(3)]

    def gather_copy(p, s):
        u = s % 2
        return pltpu.make_async_copy(
            t_sh.at[idx_v.at[p, s, 0]], seg_v.at[u], gsem[u])

    def write_copy(k, p):
        base = pl.multiple_of(base0 + k * CHUNK, 16)
        return pltpu.make_async_copy(
            rows_v.at[p], out_hbm.at[pl.ds(base, CHUNK)], wsem[p])

    def compute_idx(p):
        for g in range(NGRP):
            s, col = divmod(g * L, SEG)
            v0 = jnp.minimum(a0_v[p, 0, pl.ds(g * L, L)], N0 - 1)
            v1 = jnp.minimum(a1_v[p, 0, pl.ds(g * L, L)], N1 - 1)
            v2 = jnp.minimum(a2_v[p, 0, pl.ds(g * L, L)], N2 - 1)
            idx_v[p, s, 0, pl.ds(col, L)] = v0 * (N1 * N2) + v1 * N2 + v2

    def repack_seg(p, s):
        u = s % 2

        def row_body(r, carry):
            for k in range(NK):
                rows_v[p, s * SEG + r, pl.ds(k * L, L)] = (
                    seg_v[u, r, pl.ds(k * L, L)])
            return carry

        lax.fori_loop(0, SEG, row_body, jnp.int32(0))

    def process(k, p, prefetch_next, first_pair):
        for cp in attr_copies(k, p):
            cp.wait()
        compute_idx(p)
        if prefetch_next:
            for cp in attr_copies(k + 1, 1 - p):
                cp.start()
        if not first_pair:
            write_copy(k, p).wait()
        gather_copy(p, 0).start()
        gather_copy(p, 1).start()
        for s in range(NSEG):
            gather_copy(p, s).wait()
            repack_seg(p, s)
            if s + 2 < NSEG:
                gather_copy(p, s + 2).start()
        write_copy(k, p).start()

    for _cp in attr_copies(0, 0):
        _cp.start()

    process(0, 0, True, True)
    process(1, 1, True, True)

    def pair_body(k2, carry):
        k = 2 * k2
        process(k, 0, True, False)
        process(k + 1, 1, True, False)
        return carry

    lax.fori_loop(1, NCHUNK // 2, pair_body, jnp.int32(0))

    k_last = NCHUNK - 1
    process(k_last, 0, False, False)
    write_copy(k_last - 1, 1).wait()
    write_copy(k_last, 0).wait()


def kernel(edge_attr, W0, W1, W2):
    table = _build_table(W0, W1, W2)
    a0 = edge_attr[:, 0]
    a1 = edge_attr[:, 1]
    a2 = edge_attr[:, 2]
    return _sc_gather(a0, a1, a2, table)
